# baseline (speedup 1.0000x reference)
import functools

import jax
import jax.numpy as jnp
from jax import lax
from jax.experimental import pallas as pl
from jax.experimental.pallas import tpu as pltpu, tpu_sc as plsc

NUM_CAT = 13
D_CAT = 16
N_SUB = 5

_NC = 2
_NS = 16
_NW = _NC * _NS
_L = 16

_OFF_TABLE = 0
_OFF_W1 = 208
_OFF_B1 = 464
_OFF_W2 = 480
_OFF_B2 = 560
_W_LEN = 576


def _rsqrt_newton(ss):
    ss = jnp.maximum(ss, 1e-24)
    bits = lax.bitcast_convert_type(ss, jnp.int32)
    y = lax.bitcast_convert_type(
        jnp.int32(0x5F3759DF) - (bits >> 1), jnp.float32
    )
    for _ in range(4):
        y = y * (1.5 - 0.5 * ss * y * y)
    return y


_EXP_COEFFS = [1.0 / 40320.0, 1.0 / 5040.0, 1.0 / 720.0, 1.0 / 120.0,
               1.0 / 24.0, 1.0 / 6.0, 0.5, 1.0, 1.0]
_LOG2E = 1.4426950408889634
_LN2 = 0.6931471805599453


def _exp_precise(x):
    z = jnp.maximum(x * _LOG2E, -120.0)
    r = (z - 0.5).astype(jnp.int32)
    f = z - r.astype(jnp.float32)
    t = f * _LN2
    p = jnp.full_like(t, _EXP_COEFFS[0])
    for c in _EXP_COEFFS[1:]:
        p = p * t + c
    scale = lax.bitcast_convert_type((r + 127) << 23, jnp.float32)
    return p * scale


def _recip_precise(x):
    y = 1.0 / x
    y = y * (2.0 - x * y)
    y = y * (2.0 - x * y)
    return y


def _splat(ref, idx):
    return plsc.load_gather(ref, [jnp.full((_L,), idx, jnp.int32)])


def _make_sc_kernel(batch):
    bpw = batch // _NW
    groups = bpw // _L
    mesh = plsc.VectorSubcoreMesh(core_axis_name="c", subcore_axis_name="s")

    @functools.partial(
        pl.kernel,
        mesh=mesh,
        out_type=jax.ShapeDtypeStruct((batch * N_SUB,), jnp.float32),
        scratch_types=[
            pltpu.VMEM((_W_LEN,), jnp.float32),
            pltpu.VMEM((5 * 256,), jnp.float32),
            pltpu.VMEM((bpw,), jnp.int32),
            pltpu.VMEM((bpw,), jnp.int32),
            pltpu.VMEM((bpw * N_SUB,), jnp.float32),
        ],
        compiler_params=pltpu.CompilerParams(needs_layout_passes=False),
    )
    def sc_kernel(w_hbm, cat_hbm, tcat_hbm, out_hbm,
                  wv, pv, cat_v, tcat_v, out_v):
        wid = lax.axis_index("s") * _NC + lax.axis_index("c")
        base = wid * bpw
        pltpu.sync_copy(w_hbm, wv)
        pltpu.sync_copy(cat_hbm.at[pl.ds(base, bpw)], cat_v)
        pltpu.sync_copy(tcat_hbm.at[pl.ds(base, bpw)], tcat_v)
        lane = lax.iota(jnp.int32, _L)

        row_base = lane * D_CAT
        cols = [plsc.load_gather(wv, [row_base + k]) for k in range(D_CAT)]
        ss = cols[0] * cols[0]
        for k in range(1, D_CAT):
            ss = ss + cols[k] * cols[k]
        inv = _rsqrt_newton(ss)
        n_cols = [c * inv for c in cols]

        M = []
        Mb = []
        for j in range(D_CAT):
            acc = n_cols[0] * _splat(wv, _OFF_W1 + j)
            for k in range(1, D_CAT):
                acc = acc + n_cols[k] * _splat(wv, _OFF_W1 + k * D_CAT + j)
            M.append(acc)
            Mb.append(acc + _splat(wv, _OFF_B1 + j))

        for i in range(NUM_CAT):
            ivec = jnp.full((_L,), i, jnp.int32)
            h = []
            for j in range(D_CAT):
                mij = M[j].at[ivec].get(mode="promise_in_bounds")
                h.append(jnp.maximum(Mb[j] + mij, 0.0))
            logits = []
            for s in range(N_SUB):
                acc = h[0] * _splat(wv, _OFF_W2 + s)
                for j in range(1, D_CAT):
                    acc = acc + h[j] * _splat(wv, _OFF_W2 + j * N_SUB + s)
                logits.append(acc + _splat(wv, _OFF_B2 + s))
            m = logits[0]
            for s in range(1, N_SUB):
                m = jnp.maximum(m, logits[s])
            ex = [_exp_precise(l - m) for l in logits]
            tot = ex[0]
            for s in range(1, N_SUB):
                tot = tot + ex[s]
            inv_tot = _recip_precise(tot)
            for s in range(N_SUB):
                pv[pl.ds(s * 256 + i * _L, _L)] = ex[s] * inv_tot

        plsc.subcore_barrier()

        zero = jnp.zeros((_L,), jnp.int32)
        topc = jnp.full((_L,), NUM_CAT - 1, jnp.int32)
        for g in range(groups):
            c = cat_v[pl.ds(g * _L, _L)]
            t = tcat_v[pl.ds(g * _L, _L)]
            c = jnp.minimum(jnp.maximum(c, zero), topc)
            t = jnp.minimum(jnp.maximum(t, zero), topc)
            row = c * 16 + t
            for s in range(N_SUB):
                vals = plsc.load_gather(pv, [row + s * 256])
                plsc.store_scatter(
                    out_v, [lane * N_SUB + (g * _L * N_SUB + s)], vals)
        pltpu.sync_copy(out_v, out_hbm.at[pl.ds(base * N_SUB, bpw * N_SUB)])

    return sc_kernel


def kernel(category, target_category, table, W1, b1, W2, b2):
    f32 = jnp.float32
    batch = category.shape[0]
    wcat = jnp.concatenate([
        table.astype(f32).reshape(-1),
        W1.astype(f32).reshape(-1),
        b1.astype(f32),
        W2.astype(f32).reshape(-1),
        b2.astype(f32),
        jnp.zeros((_W_LEN - 565,), f32),
    ])
    cat = category.astype(jnp.int32)
    tcat = target_category.astype(jnp.int32)
    out_flat = _make_sc_kernel(batch)(wcat, cat, tcat)
    return out_flat.reshape(batch, N_SUB)


# speedup vs baseline: 1.2257x; 1.2257x over previous
import functools

import jax
import jax.numpy as jnp
from jax import lax
from jax.experimental import pallas as pl
from jax.experimental.pallas import tpu as pltpu, tpu_sc as plsc

NUM_CAT = 13
D_CAT = 16
N_SUB = 5

_NC = 2
_NS = 16
_NW = _NC * _NS
_L = 16

_PV_LEN = N_SUB * 256


def _pair_table_body(t_ref, w1_ref, b1_ref, w2_ref, b2_ref, p_ref):
    t13 = t_ref[:]
    t = jnp.concatenate([t13, jnp.zeros((3, 16), jnp.float32)], axis=0)
    norm = jnp.sqrt(jnp.sum(t * t, axis=1, keepdims=True))
    n = t / jnp.maximum(norm, 1e-12)
    w1 = w1_ref[:]
    b1 = jnp.reshape(b1_ref[:], (1, 16))
    w2 = jnp.concatenate(
        [w2_ref[:], jnp.zeros((16, 11), jnp.float32)], axis=1
    )
    b2 = jnp.reshape(
        jnp.concatenate([b2_ref[:], jnp.full((11,), -1e30, jnp.float32)]),
        (1, 16),
    )
    for i in range(16):
        e = n + n[i : i + 1, :]
        h = jnp.maximum(
            jnp.dot(e, w1, preferred_element_type=jnp.float32) + b1, 0.0
        )
        g = jnp.dot(h, w2, preferred_element_type=jnp.float32) + b2
        m = jnp.max(g, axis=1, keepdims=True)
        ex = jnp.exp(g - m)
        sm = ex / jnp.sum(ex, axis=1, keepdims=True)
        p_ref[:, pl.ds(i * 16, 16)] = sm.T


def _make_sc_lookup(batch):
    bpw = batch // _NW
    groups = bpw // _L
    mesh = plsc.VectorSubcoreMesh(core_axis_name="c", subcore_axis_name="s")

    @functools.partial(
        pl.kernel,
        mesh=mesh,
        out_type=jax.ShapeDtypeStruct((batch * N_SUB,), jnp.float32),
        scratch_types=[
            pltpu.VMEM((_PV_LEN,), jnp.float32),
            pltpu.VMEM((bpw,), jnp.int32),
            pltpu.VMEM((bpw,), jnp.int32),
            pltpu.VMEM((bpw * N_SUB,), jnp.float32),
            pltpu.SemaphoreType.DMA,
            pltpu.SemaphoreType.DMA,
            pltpu.SemaphoreType.DMA,
        ],
        compiler_params=pltpu.CompilerParams(needs_layout_passes=False),
    )
    def sc_lookup(p_hbm, cat_hbm, tcat_hbm, out_hbm,
                  p_v, cat_v, tcat_v, out_v, sem0, sem1, sem2):
        wid = lax.axis_index("s") * _NC + lax.axis_index("c")
        base = wid * bpw
        c0 = pltpu.async_copy(p_hbm.at[pl.ds(0, _PV_LEN)], p_v, sem0)
        c1 = pltpu.async_copy(cat_hbm.at[pl.ds(base, bpw)], cat_v, sem1)
        c2 = pltpu.async_copy(tcat_hbm.at[pl.ds(base, bpw)], tcat_v, sem2)
        c0.wait()
        c1.wait()
        c2.wait()
        lane = lax.iota(jnp.int32, _L)
        zero = jnp.zeros((_L,), jnp.int32)
        topc = jnp.full((_L,), NUM_CAT - 1, jnp.int32)
        for g in range(groups):
            c = cat_v[pl.ds(g * _L, _L)]
            t = tcat_v[pl.ds(g * _L, _L)]
            c = jnp.minimum(jnp.maximum(c, zero), topc)
            t = jnp.minimum(jnp.maximum(t, zero), topc)
            row = c * 16 + t
            for s in range(N_SUB):
                vals = plsc.load_gather(p_v, [row + s * 256])
                plsc.store_scatter(
                    out_v, [lane * N_SUB + (g * _L * N_SUB + s)], vals)
        pltpu.sync_copy(out_v, out_hbm.at[pl.ds(base * N_SUB, bpw * N_SUB)])

    return sc_lookup


def kernel(category, target_category, table, W1, b1, W2, b2):
    f32 = jnp.float32
    batch = category.shape[0]

    pair_table = pl.pallas_call(
        _pair_table_body,
        out_shape=jax.ShapeDtypeStruct((16, 256), f32),
    )(table.astype(f32), W1.astype(f32), b1.astype(f32), W2.astype(f32),
      b2.astype(f32))

    cat = category.astype(jnp.int32)
    tcat = target_category.astype(jnp.int32)
    out_flat = _make_sc_lookup(batch)(pair_table.reshape(4096), cat, tcat)
    return out_flat.reshape(batch, N_SUB)


# speedup vs baseline: 2.0578x; 1.6788x over previous
import functools

import jax
import jax.numpy as jnp
from jax import lax
from jax.experimental import pallas as pl
from jax.experimental.pallas import tpu as pltpu, tpu_sc as plsc

NUM_CAT = 13
D_CAT = 16
N_SUB = 5

_NC = 2
_NS = 16
_NW = _NC * _NS
_L = 16

_PV_LEN = N_SUB * 256


def _pair_table_body(t_ref, w1_ref, b1_ref, w2_ref, b2_ref, p_ref):
    t13 = t_ref[:]
    t = jnp.concatenate([t13, jnp.zeros((3, 16), jnp.float32)], axis=0)
    norm = jnp.sqrt(jnp.sum(t * t, axis=1, keepdims=True))
    n = t / jnp.maximum(norm, 1e-12)
    w1 = w1_ref[:]
    b1 = jnp.reshape(b1_ref[:], (1, 16))
    w2 = jnp.concatenate(
        [w2_ref[:], jnp.zeros((16, 11), jnp.float32)], axis=1
    )
    b2 = jnp.reshape(
        jnp.concatenate([b2_ref[:], jnp.full((11,), -1e30, jnp.float32)]),
        (1, 16),
    )
    for i in range(16):
        e = n + n[i : i + 1, :]
        h = jnp.maximum(
            jnp.dot(e, w1, preferred_element_type=jnp.float32) + b1, 0.0
        )
        g = jnp.dot(h, w2, preferred_element_type=jnp.float32) + b2
        m = jnp.max(g, axis=1, keepdims=True)
        ex = jnp.exp(g - m)
        sm = ex / jnp.sum(ex, axis=1, keepdims=True)
        p_ref[:, pl.ds(i * 16, 16)] = sm.T


def _make_sc_lookup(batch):
    bpw = batch // _NW
    groups = bpw // _L
    mesh = plsc.VectorSubcoreMesh(core_axis_name="c", subcore_axis_name="s")

    @functools.partial(
        pl.kernel,
        mesh=mesh,
        out_type=jax.ShapeDtypeStruct((N_SUB, batch), jnp.float32),
        scratch_types=[
            pltpu.VMEM((_PV_LEN,), jnp.float32),
            pltpu.VMEM((bpw,), jnp.int32),
            pltpu.VMEM((bpw,), jnp.int32),
            pltpu.VMEM((N_SUB, bpw), jnp.float32),
            pltpu.SemaphoreType.DMA,
            pltpu.SemaphoreType.DMA,
            pltpu.SemaphoreType.DMA,
        ],
        compiler_params=pltpu.CompilerParams(needs_layout_passes=False),
    )
    def sc_lookup(p_hbm, cat_hbm, tcat_hbm, out_hbm,
                  p_v, cat_v, tcat_v, out_v, sem0, sem1, sem2):
        wid = lax.axis_index("s") * _NC + lax.axis_index("c")
        base = wid * bpw
        c0 = pltpu.async_copy(p_hbm.at[pl.ds(0, _PV_LEN)], p_v, sem0)
        c1 = pltpu.async_copy(cat_hbm.at[pl.ds(base, bpw)], cat_v, sem1)
        c2 = pltpu.async_copy(tcat_hbm.at[pl.ds(base, bpw)], tcat_v, sem2)
        c0.wait()
        c1.wait()
        c2.wait()
        zero = jnp.zeros((_L,), jnp.int32)
        topc = jnp.full((_L,), NUM_CAT - 1, jnp.int32)
        for g in range(groups):
            c = cat_v[pl.ds(g * _L, _L)]
            t = tcat_v[pl.ds(g * _L, _L)]
            c = jnp.minimum(jnp.maximum(c, zero), topc)
            t = jnp.minimum(jnp.maximum(t, zero), topc)
            row = c * 16 + t
            for s in range(N_SUB):
                vals = plsc.load_gather(p_v, [row + s * 256])
                out_v[s, pl.ds(g * _L, _L)] = vals
        pltpu.sync_copy(out_v, out_hbm.at[:, pl.ds(base, bpw)])

    return sc_lookup


def kernel(category, target_category, table, W1, b1, W2, b2):
    f32 = jnp.float32
    batch = category.shape[0]

    pair_table = pl.pallas_call(
        _pair_table_body,
        out_shape=jax.ShapeDtypeStruct((16, 256), f32),
    )(table.astype(f32), W1.astype(f32), b1.astype(f32), W2.astype(f32),
      b2.astype(f32))

    cat = category.astype(jnp.int32)
    tcat = target_category.astype(jnp.int32)
    out_t = _make_sc_lookup(batch)(pair_table.reshape(4096), cat, tcat)
    return out_t.T


# speedup vs baseline: 2.0759x; 1.0088x over previous
import functools

import jax
import jax.numpy as jnp
from jax import lax
from jax.experimental import pallas as pl
from jax.experimental.pallas import tpu as pltpu, tpu_sc as plsc

NUM_CAT = 13
D_CAT = 16
N_SUB = 5

_NC = 2
_NS = 16
_NW = _NC * _NS
_L = 16

_PV_LEN = N_SUB * 256


def _pair_table_body(t_ref, w1_ref, b1_ref, w2_ref, b2_ref, p_ref):
    t13 = t_ref[:]
    t = jnp.concatenate([t13, jnp.zeros((3, 16), jnp.float32)], axis=0)
    norm = jnp.sqrt(jnp.sum(t * t, axis=1, keepdims=True))
    n = t / jnp.maximum(norm, 1e-12)
    w1 = w1_ref[:]
    b1 = jnp.reshape(b1_ref[:], (1, 16))
    w2 = jnp.concatenate(
        [w2_ref[:], jnp.zeros((16, 11), jnp.float32)], axis=1
    )
    b2 = jnp.reshape(
        jnp.concatenate([b2_ref[:], jnp.full((11,), -1e30, jnp.float32)]),
        (1, 16),
    )
    for i in range(16):
        e = n + n[i : i + 1, :]
        h = jnp.maximum(
            jnp.dot(e, w1, preferred_element_type=jnp.float32) + b1, 0.0
        )
        g = jnp.dot(h, w2, preferred_element_type=jnp.float32) + b2
        m = jnp.max(g, axis=1, keepdims=True)
        ex = jnp.exp(g - m)
        sm = ex / jnp.sum(ex, axis=1, keepdims=True)
        p_ref[:, pl.ds(i * 16, 16)] = sm.T


def _make_sc_lookup(batch):
    bpw = batch // _NW
    groups = bpw // _L
    mesh = plsc.VectorSubcoreMesh(core_axis_name="c", subcore_axis_name="s")

    @functools.partial(
        pl.kernel,
        mesh=mesh,
        out_type=jax.ShapeDtypeStruct((N_SUB, batch), jnp.float32),
        scratch_types=[
            pltpu.VMEM((_PV_LEN,), jnp.float32),
            pltpu.VMEM((bpw,), jnp.int32),
            pltpu.VMEM((bpw,), jnp.int32),
            pltpu.VMEM((N_SUB, bpw), jnp.float32),
            pltpu.SemaphoreType.DMA,
            pltpu.SemaphoreType.DMA,
            pltpu.SemaphoreType.DMA,
        ],
        compiler_params=pltpu.CompilerParams(needs_layout_passes=False),
    )
    def sc_lookup(p_hbm, cat_hbm, tcat_hbm, out_hbm,
                  p_v, cat_v, tcat_v, out_v, sem0, sem1, sem2):
        wid = lax.axis_index("s") * _NC + lax.axis_index("c")
        base = wid * bpw
        c0 = pltpu.async_copy(p_hbm.at[pl.ds(0, _PV_LEN)], p_v, sem0)
        c1 = pltpu.async_copy(cat_hbm.at[pl.ds(base, bpw)], cat_v, sem1)
        c2 = pltpu.async_copy(tcat_hbm.at[pl.ds(base, bpw)], tcat_v, sem2)
        c0.wait()
        c1.wait()
        c2.wait()
        zero = jnp.zeros((_L,), jnp.int32)
        topc = jnp.full((_L,), NUM_CAT - 1, jnp.int32)

        def body(g, carry):
            off = g * _L
            c = cat_v[pl.ds(off, _L)]
            t = tcat_v[pl.ds(off, _L)]
            c = jnp.minimum(jnp.maximum(c, zero), topc)
            t = jnp.minimum(jnp.maximum(t, zero), topc)
            row = c * 16 + t
            for s in range(N_SUB):
                vals = plsc.load_gather(p_v, [row + s * 256])
                out_v[s, pl.ds(off, _L)] = vals
            return carry

        lax.fori_loop(0, groups, body, 0)
        pltpu.sync_copy(out_v, out_hbm.at[:, pl.ds(base, bpw)])

    return sc_lookup


def kernel(category, target_category, table, W1, b1, W2, b2):
    f32 = jnp.float32
    batch = category.shape[0]

    pair_table = pl.pallas_call(
        _pair_table_body,
        out_shape=jax.ShapeDtypeStruct((16, 256), f32),
    )(table.astype(f32), W1.astype(f32), b1.astype(f32), W2.astype(f32),
      b2.astype(f32))

    cat = category.astype(jnp.int32)
    tcat = target_category.astype(jnp.int32)
    out_t = _make_sc_lookup(batch)(pair_table.reshape(4096), cat, tcat)
    return out_t.T


# speedup vs baseline: 2.1225x; 1.0225x over previous
import functools

import jax
import jax.numpy as jnp
from jax import lax
from jax.experimental import pallas as pl
from jax.experimental.pallas import tpu as pltpu, tpu_sc as plsc

NUM_CAT = 13
D_CAT = 16
N_SUB = 5

_NC = 2
_NS = 16
_NW = _NC * _NS
_L = 16

_OFF_TABLE = 0
_OFF_W1 = 208
_OFF_B1 = 464
_OFF_W2 = 480
_OFF_B2 = 560
_W_LEN = 576

_BLK = N_SUB * _L
_PV_LEN = NUM_CAT * _BLK

_EXP_COEFFS = [1.0 / 40320.0, 1.0 / 5040.0, 1.0 / 720.0, 1.0 / 120.0,
               1.0 / 24.0, 1.0 / 6.0, 0.5, 1.0, 1.0]
_LOG2E = 1.4426950408889634
_LN2 = 0.6931471805599453


def _exp_precise(x):
    z = jnp.maximum(x * _LOG2E, -120.0)
    r = (z - 0.5).astype(jnp.int32)
    f = z - r.astype(jnp.float32)
    t = f * _LN2
    p = jnp.full_like(t, _EXP_COEFFS[0])
    for c in _EXP_COEFFS[1:]:
        p = p * t + c
    scale = lax.bitcast_convert_type((r + 127) << 23, jnp.float32)
    return p * scale


def _recip_precise(x):
    y = 1.0 / x
    y = y * (2.0 - x * y)
    y = y * (2.0 - x * y)
    return y


def _rsqrt_newton(ss):
    ss = jnp.maximum(ss, 1e-24)
    bits = lax.bitcast_convert_type(ss, jnp.int32)
    y = lax.bitcast_convert_type(
        jnp.int32(0x5F3759DF) - (bits >> 1), jnp.float32
    )
    for _ in range(4):
        y = y * (1.5 - 0.5 * ss * y * y)
    return y


def _splat(ref, idx):
    return plsc.load_gather(ref, [jnp.full((_L,), idx, jnp.int32)])


def _make_sc_kernel(batch):
    bpw = batch // _NW
    groups = bpw // _L
    mesh = plsc.VectorSubcoreMesh(core_axis_name="c", subcore_axis_name="s")

    @functools.partial(
        pl.kernel,
        mesh=mesh,
        out_type=jax.ShapeDtypeStruct((N_SUB, batch), jnp.float32),
        scratch_types=[
            pltpu.VMEM((_W_LEN,), jnp.float32),
            pltpu.VMEM((_BLK,), jnp.float32),
            pltpu.VMEM((_PV_LEN,), jnp.float32),
            pltpu.VMEM((bpw,), jnp.int32),
            pltpu.VMEM((bpw,), jnp.int32),
            pltpu.VMEM((N_SUB, bpw), jnp.float32),
            pltpu.VMEM_SHARED((_PV_LEN,), jnp.float32),
            pltpu.SemaphoreType.DMA,
            pltpu.SemaphoreType.DMA,
            pltpu.SemaphoreType.DMA,
        ],
        compiler_params=pltpu.CompilerParams(needs_layout_passes=False),
    )
    def sc_kernel(w_hbm, cat_hbm, tcat_hbm, out_hbm,
                  wv, blk_v, pv, cat_v, tcat_v, out_v, shared,
                  sem0, sem1, sem2):
        sid = lax.axis_index("s")
        wid = sid * _NC + lax.axis_index("c")
        base = wid * bpw
        c0 = pltpu.async_copy(w_hbm, wv, sem0)
        c1 = pltpu.async_copy(cat_hbm.at[pl.ds(base, bpw)], cat_v, sem1)
        c2 = pltpu.async_copy(tcat_hbm.at[pl.ds(base, bpw)], tcat_v, sem2)
        c0.wait()
        lane = lax.iota(jnp.int32, _L)

        row_base = lane * D_CAT
        cols = [plsc.load_gather(wv, [row_base + k]) for k in range(D_CAT)]
        ss = cols[0] * cols[0]
        for k in range(1, D_CAT):
            ss = ss + cols[k] * cols[k]
        inv = _rsqrt_newton(ss)
        n_cols = [c * inv for c in cols]

        M = []
        Mb = []
        for j in range(D_CAT):
            acc = n_cols[0] * _splat(wv, _OFF_W1 + j)
            for k in range(1, D_CAT):
                acc = acc + n_cols[k] * _splat(wv, _OFF_W1 + k * D_CAT + j)
            M.append(acc)
            Mb.append(acc + _splat(wv, _OFF_B1 + j))

        @pl.when(sid < NUM_CAT)
        def _compute_block():
            ivec = jnp.full((_L,), 0, jnp.int32) + sid
            h = []
            for j in range(D_CAT):
                mij = M[j].at[ivec].get(mode="promise_in_bounds")
                h.append(jnp.maximum(Mb[j] + mij, 0.0))
            logits = []
            for s in range(N_SUB):
                acc = h[0] * _splat(wv, _OFF_W2 + s)
                for j in range(1, D_CAT):
                    acc = acc + h[j] * _splat(wv, _OFF_W2 + j * N_SUB + s)
                logits.append(acc + _splat(wv, _OFF_B2 + s))
            m = logits[0]
            for s in range(1, N_SUB):
                m = jnp.maximum(m, logits[s])
            ex = [_exp_precise(l - m) for l in logits]
            tot = ex[0]
            for s in range(1, N_SUB):
                tot = tot + ex[s]
            inv_tot = _recip_precise(tot)
            for s in range(N_SUB):
                blk_v[pl.ds(s * _L, _L)] = ex[s] * inv_tot
            pltpu.sync_copy(blk_v, shared.at[pl.ds(sid * _BLK, _BLK)])

        plsc.subcore_barrier()
        pltpu.sync_copy(shared, pv)
        c1.wait()
        c2.wait()

        zero = jnp.zeros((_L,), jnp.int32)
        topc = jnp.full((_L,), NUM_CAT - 1, jnp.int32)

        def body(g, carry):
            off = g * _L
            c = cat_v[pl.ds(off, _L)]
            t = tcat_v[pl.ds(off, _L)]
            c = jnp.minimum(jnp.maximum(c, zero), topc)
            t = jnp.minimum(jnp.maximum(t, zero), topc)
            idx = c * _BLK + t
            for s in range(N_SUB):
                vals = plsc.load_gather(pv, [idx + s * _L])
                out_v[s, pl.ds(off, _L)] = vals
            return carry

        lax.fori_loop(0, groups, body, 0)
        pltpu.sync_copy(out_v, out_hbm.at[:, pl.ds(base, bpw)])

    return sc_kernel


def kernel(category, target_category, table, W1, b1, W2, b2):
    f32 = jnp.float32
    batch = category.shape[0]
    wcat = jnp.concatenate([
        table.astype(f32).reshape(-1),
        W1.astype(f32).reshape(-1),
        b1.astype(f32),
        W2.astype(f32).reshape(-1),
        b2.astype(f32),
        jnp.zeros((_W_LEN - 565,), f32),
    ])
    cat = category.astype(jnp.int32)
    tcat = target_category.astype(jnp.int32)
    out_t = _make_sc_kernel(batch)(wcat, cat, tcat)
    return out_t.T
